# baseline (speedup 1.0000x reference)
import functools

import jax
import jax.numpy as jnp
from jax import lax
from jax.experimental import pallas as pl
from jax.experimental.pallas import tpu as pltpu
from jax.experimental.pallas import tpu_sc as plsc

VOCAB = 1000000
DIM = 64
B = 16384
NEG = 10
K1 = NEG + 1
R = K1 + 1

_info = plsc.get_sparse_core_info()
NC, NS, LANES = _info.num_cores, _info.num_subcores, _info.num_lanes
NW = NC * NS
BPW = B // NW
CHUNK = 32
NCH = BPW // CHUNK
NGRP = CHUNK // LANES
RB = R * CHUNK
IPW = NCH * RB

_mesh = plsc.VectorSubcoreMesh(core_axis_name="c", subcore_axis_name="s")


@functools.partial(
    pl.kernel,
    out_type=jax.ShapeDtypeStruct((K1 * B,), jnp.float32),
    mesh=_mesh,
    scratch_types=[
        pltpu.VMEM((IPW,), jnp.int32),
        pltpu.VMEM((4, RB, DIM), jnp.bfloat16),
        pltpu.VMEM((K1 * BPW,), jnp.float32),
        pltpu.SemaphoreType.DMA,
        pltpu.SemaphoreType.DMA,
        pltpu.SemaphoreType.DMA,
        pltpu.SemaphoreType.DMA,
    ],
    compiler_params=pltpu.CompilerParams(
        needs_layout_passes=False, use_tc_tiling_on_sc=False),
)
def _sc_scores(cen_tab, ctx_tab, all_idx, out,
               idx_v, rows_v, scores_v, sem_a, sem_b, sem_c, sem_d):
    wid = lax.axis_index("s") * NC + lax.axis_index("c")

    pltpu.sync_copy(all_idx.at[pl.ds(wid * IPW, IPW)], idx_v)

    sems = [sem_a, sem_b, sem_c, sem_d]

    def issue(c):
        buf = c % 4
        off = c * RB
        sem = sems[buf]
        hs = [pltpu.async_copy(
            cen_tab.at[idx_v.at[pl.ds(off, CHUNK)]],
            rows_v.at[buf, pl.ds(0, CHUNK)], sem)]
        hs.append(pltpu.async_copy(
            ctx_tab.at[idx_v.at[pl.ds(off + CHUNK, K1 * CHUNK)]],
            rows_v.at[buf, pl.ds(CHUNK, K1 * CHUNK)], sem))
        return hs

    lane = lax.iota(jnp.int32, LANES)
    NQ = DIM // LANES

    def compute(c):
        buf = c % 4

        def group(g, _):
            def body(i, acc):
                r = g * LANES + i

                def halves(rr):
                    hs = []
                    for q in range(NQ // 2):
                        x = rows_v[buf, rr, pl.ds(q * 2 * LANES, 2 * LANES)]
                        hs.extend(plsc.unpack(
                            x, format=plsc.PackFormat.INTERLEAVED,
                            preferred_element_type=jnp.float32))
                    return hs

                cen = halves(r)
                out = []
                for j in range(K1):
                    ctx = halves((j + 1) * CHUNK + r)
                    v = cen[0] * ctx[0]
                    for q in range(1, len(cen)):
                        v = v + cen[q] * ctx[q]
                    s = jnp.sum(v)
                    out.append(jnp.where(lane == i,
                                         jnp.full((LANES,), 1.0,
                                                  jnp.float32) * s,
                                         acc[j]))
                return tuple(out)

            accs = plsc.parallel_loop(
                0, LANES, unroll=2,
                carry=tuple(jnp.zeros((LANES,), jnp.float32)
                            for _ in range(K1)))(body)

            row = c * CHUNK + g * LANES
            for j in range(K1):
                scores_v[pl.ds(j * BPW + row, LANES)] = accs[j]
            return 0

        lax.fori_loop(0, NGRP, group, 0)

    pending = [issue(0), issue(1), issue(2)]
    for c in range(NCH):
        if c + 3 < NCH:
            pending.append(issue(c + 3))
        for h in pending.pop(0):
            h.wait()
        compute(c)

    wbase = wid * BPW
    for j in range(K1):
        pltpu.sync_copy(scores_v.at[pl.ds(j * BPW, BPW)],
                        out.at[pl.ds(j * B + wbase, BPW)])


def _loss_body(s_ref, o_ref):
    x = jnp.clip(s_ref[...], -10.0, 10.0)
    row = lax.broadcasted_iota(jnp.int32, (K1, B), 0)
    y = jnp.where(row == 0, x, -x)
    ll = -jnp.log1p(jnp.exp(-y))
    o_ref[...] = jnp.reshape(-jnp.sum(ll) / B, (1, 1))


def kernel(center_word, context_word, neg_words, center_table, context_table):
    all_idx = jnp.concatenate(
        [center_word[None, :], context_word[None, :], neg_words.T],
        axis=0).astype(jnp.int32)
    all_idx = all_idx.reshape(R, NW, NCH, CHUNK).transpose(1, 2, 0, 3)
    scores = _sc_scores(center_table.astype(jnp.bfloat16),
                        context_table.astype(jnp.bfloat16),
                        all_idx.reshape(R * B))
    loss = pl.pallas_call(
        _loss_body,
        out_shape=jax.ShapeDtypeStruct((1, 1), jnp.float32),
    )(scores.reshape(K1, B))
    return loss[0, 0]


# speedup vs baseline: 1.3116x; 1.3116x over previous
import functools

import jax
import jax.numpy as jnp
from jax import lax
from jax.experimental import pallas as pl
from jax.experimental.pallas import tpu as pltpu
from jax.experimental.pallas import tpu_sc as plsc

VOCAB = 1000000
DIM = 64
B = 16384
NEG = 10
K1 = NEG + 1
R = K1 + 1

_info = plsc.get_sparse_core_info()
NC, NS, LANES = _info.num_cores, _info.num_subcores, _info.num_lanes
NW = NC * NS
BPW = B // NW
CHUNK = 64
NCH = BPW // CHUNK
NGRP = CHUNK // LANES
RB = R * CHUNK
IPW = NCH * RB

_mesh = plsc.VectorSubcoreMesh(core_axis_name="c", subcore_axis_name="s")


@functools.partial(
    pl.kernel,
    out_type=jax.ShapeDtypeStruct((K1 * B,), jnp.float32),
    mesh=_mesh,
    scratch_types=[
        pltpu.VMEM((IPW,), jnp.int32),
        pltpu.VMEM((2, RB, DIM), jnp.float32),
        pltpu.VMEM((K1 * BPW,), jnp.float32),
        pltpu.SemaphoreType.DMA,
        pltpu.SemaphoreType.DMA,
    ],
    compiler_params=pltpu.CompilerParams(
        needs_layout_passes=False, use_tc_tiling_on_sc=False),
)
def _sc_scores(cen_tab, ctx_tab, all_idx, out,
               idx_v, rows_v, scores_v, sem_a, sem_b):
    wid = lax.axis_index("s") * NC + lax.axis_index("c")

    pltpu.sync_copy(all_idx.at[pl.ds(wid * IPW, IPW)], idx_v)

    def issue(c):
        buf = c % 2
        off = c * RB
        sem = sem_a if buf == 0 else sem_b
        hs = [pltpu.async_copy(
            cen_tab.at[idx_v.at[pl.ds(off, CHUNK)]],
            rows_v.at[buf, pl.ds(0, CHUNK)], sem)]
        hs.append(pltpu.async_copy(
            ctx_tab.at[idx_v.at[pl.ds(off + CHUNK, K1 * CHUNK)]],
            rows_v.at[buf, pl.ds(CHUNK, K1 * CHUNK)], sem))
        return hs

    lane = lax.iota(jnp.int32, LANES)
    NQ = DIM // LANES

    def compute(c):
        buf = c % 2

        def group(g, _):
            def body(i, acc):
                r = g * LANES + i
                cen = [rows_v[buf, r, pl.ds(q * LANES, LANES)]
                       for q in range(NQ)]
                out = []
                for j in range(K1):
                    rj = (j + 1) * CHUNK + r
                    v = cen[0] * rows_v[buf, rj, pl.ds(0, LANES)]
                    for q in range(1, NQ):
                        v = v + cen[q] * rows_v[buf, rj,
                                                pl.ds(q * LANES, LANES)]
                    s = jnp.sum(v)
                    out.append(jnp.where(lane == i,
                                         jnp.full((LANES,), 1.0,
                                                  jnp.float32) * s,
                                         acc[j]))
                return tuple(out)

            accs = plsc.parallel_loop(
                0, LANES, unroll=2,
                carry=tuple(jnp.zeros((LANES,), jnp.float32)
                            for _ in range(K1)))(body)

            row = c * CHUNK + g * LANES
            for j in range(K1):
                scores_v[pl.ds(j * BPW + row, LANES)] = accs[j]
            return 0

        lax.fori_loop(0, NGRP, group, 0)

    pending = issue(0)
    for c in range(NCH):
        nxt = issue(c + 1) if c + 1 < NCH else None
        for h in pending:
            h.wait()
        compute(c)
        pending = nxt

    wbase = wid * BPW
    for j in range(K1):
        pltpu.sync_copy(scores_v.at[pl.ds(j * BPW, BPW)],
                        out.at[pl.ds(j * B + wbase, BPW)])


def _loss_body(s_ref, o_ref):
    x = jnp.clip(s_ref[...], -10.0, 10.0)
    row = lax.broadcasted_iota(jnp.int32, (K1, B), 0)
    y = jnp.where(row == 0, x, -x)
    ll = -jnp.log1p(jnp.exp(-y))
    o_ref[...] = jnp.reshape(-jnp.sum(ll) / B, (1, 1))


def kernel(center_word, context_word, neg_words, center_table, context_table):
    all_idx = jnp.concatenate(
        [center_word[None, :], context_word[None, :], neg_words.T],
        axis=0).astype(jnp.int32)
    all_idx = all_idx.reshape(R, NW, NCH, CHUNK).transpose(1, 2, 0, 3)
    scores = _sc_scores(center_table, context_table,
                        all_idx.reshape(R * B))
    loss = pl.pallas_call(
        _loss_body,
        out_shape=jax.ShapeDtypeStruct((1, 1), jnp.float32),
    )(scores.reshape(K1, B))
    return loss[0, 0]
